# Initial kernel scaffold; baseline (speedup 1.0000x reference)
import jax
import jax.numpy as jnp
from jax.experimental import pallas as pl


def kernel(output, target):
    raise NotImplementedError("write your pallas kernel here")


# speedup vs baseline: 1.0362x; 1.0362x over previous
import jax
import jax.numpy as jnp
from jax.experimental import pallas as pl
from jax.experimental.pallas import tpu as pltpu

_POS_W = 2.0
_ROWS = 128
_COLS = 32768
_BLOCK_ROWS = 16


def _reduce_body(out_ref, tgt_ref, acc_ref):
    i = pl.program_id(0)
    o = out_ref[...]
    t = tgt_ref[...]
    w = jnp.where(t > 0, -_POS_W * o, jnp.where(t < 0, o, 0.0))
    p = jnp.sum(w)

    @pl.when(i == 0)
    def _():
        acc_ref[0, 0] = 0.0

    acc_ref[0, 0] += p


def kernel(output, target):
    grid = (_ROWS // _BLOCK_ROWS,)
    res = pl.pallas_call(
        _reduce_body,
        grid=grid,
        in_specs=[
            pl.BlockSpec((_BLOCK_ROWS, _COLS), lambda i: (i, 0)),
            pl.BlockSpec((_BLOCK_ROWS, _COLS), lambda i: (i, 0)),
        ],
        out_specs=pl.BlockSpec(
            (1, 1), lambda i: (0, 0), memory_space=pltpu.SMEM
        ),
        out_shape=jax.ShapeDtypeStruct((1, 1), jnp.float32),
    )(output, target)
    return res[0, 0]
